# Initial kernel scaffold; baseline (speedup 1.0000x reference)
import jax
import jax.numpy as jnp
from jax.experimental import pallas as pl


def kernel(x, edge_index, W_res, b_res, W1, as1, ad1, bconv1, g1, be1, W2, as2, ad2, bconv2, g2, be2, Wc1, bc1, gcn, bcn, Wc2, bc2):
    raise NotImplementedError("write your pallas kernel here")


# speedup vs baseline: 1.0730x; 1.0730x over previous
import jax
import jax.numpy as jnp
from jax.experimental import pallas as pl
from jax.experimental.pallas import tpu as pltpu

N = 10000
HEADS = 8
HD = 16
HID = HEADS * HD
OUT = 64


def _bn(x, g, b):
    m = x.mean(axis=0)
    v = x.var(axis=0)
    return (x - m) / jnp.sqrt(v + 1e-5) * g + b


def _gat(x, edge_index, W, a_src, a_dst, bias):
    ar = jnp.arange(N, dtype=edge_index.dtype)
    src = jnp.concatenate([edge_index[0], ar])
    dst = jnp.concatenate([edge_index[1], ar])
    h = (x @ W.T).reshape(N, HEADS, HD)
    asrc = (h * a_src[None]).sum(-1)
    adst = (h * a_dst[None]).sum(-1)
    e = asrc[src] + adst[dst]
    e = jax.nn.leaky_relu(e, 0.2)
    ex = jnp.exp(e)
    den = jax.ops.segment_sum(ex, dst, num_segments=N)
    num = jax.ops.segment_sum(ex[:, :, None] * h[src], dst, num_segments=N)
    out = num / (den[:, :, None] + 1e-16)
    return out.reshape(N, HEADS * HD) + bias


def _classifier_body(h2_ref, wc1_ref, bc1_ref, gcn_ref, bcn_ref, wc2_ref,
                     bc2_ref, out_ref):
    c = jnp.dot(h2_ref[...], wc1_ref[...].T,
                preferred_element_type=jnp.float32) + bc1_ref[...]
    m = c.mean(axis=0, keepdims=True)
    v = jnp.mean((c - m) ** 2, axis=0, keepdims=True)
    c = (c - m) / jnp.sqrt(v + 1e-5) * gcn_ref[...] + bcn_ref[...]
    c = jnp.maximum(c, 0.0)
    logits = jnp.dot(c, wc2_ref[...].T,
                     preferred_element_type=jnp.float32) + bc2_ref[...]
    mx = jnp.max(logits, axis=1, keepdims=True)
    s = logits - mx
    lse = jnp.log(jnp.sum(jnp.exp(s), axis=1, keepdims=True))
    out_ref[...] = s - lse


def _classifier(h2, Wc1, bc1, gcn, bcn, Wc2, bc2):
    return pl.pallas_call(
        _classifier_body,
        out_shape=jax.ShapeDtypeStruct((N, OUT), jnp.float32),
    )(h2, Wc1, bc1.reshape(1, -1), gcn.reshape(1, -1), bcn.reshape(1, -1),
      Wc2, bc2.reshape(1, -1))


def kernel(x, edge_index, W_res, b_res, W1, as1, ad1, bconv1, g1, be1,
           W2, as2, ad2, bconv2, g2, be2, Wc1, bc1, gcn, bcn, Wc2, bc2):
    x_p = x @ W_res.T + b_res
    h1 = jax.nn.elu(_bn(_gat(x_p, edge_index, W1, as1, ad1, bconv1), g1, be1))
    h2 = jax.nn.elu(_bn(_gat(h1, edge_index, W2, as2, ad2, bconv2), g2, be2)) + x_p
    return _classifier(h2, Wc1, bc1, gcn, bcn, Wc2, bc2)


# speedup vs baseline: 71.2797x; 66.4325x over previous
import functools

import jax
import jax.numpy as jnp
from jax import lax
from jax.experimental import pallas as pl
from jax.experimental.pallas import tpu as pltpu
from jax.experimental.pallas import tpu_sc as plsc

N = 10000
E = 320000
HEADS = 8
HD = 16
HID = HEADS * HD
OUT = 64
TS = 144

_NC = 2
_NS = 16
_NW = _NC * _NS
_EW = E // _NW
_CH = 80
_NCHUNK = _EW // _CH
_RPT = 624
_TAIL = N - _NS * _RPT


def _prep_body(x_ref, wres_ref, bres_ref, w1_ref, asm_ref, adm_ref,
               xp_ref, tsrc_ref, tdst_ref):
    f32 = jnp.float32
    xp = jnp.dot(x_ref[...], wres_ref[...].T, preferred_element_type=f32)
    xp = xp + bres_ref[...]
    xp_ref[...] = xp
    h = jnp.dot(xp, w1_ref[...].T, preferred_element_type=f32)
    asrc = jnp.dot(h, asm_ref[...], preferred_element_type=f32)
    adst = jnp.dot(h, adm_ref[...], preferred_element_type=f32)
    z8 = jnp.zeros((N, 8), f32)
    tsrc_ref[...] = jnp.concatenate([h, asrc, z8], axis=1)
    tdst_ref[...] = jnp.concatenate([adst, z8], axis=1)


def _prep(x, W_res, b_res, W1, As, Ad):
    return pl.pallas_call(
        _prep_body,
        out_shape=(
            jax.ShapeDtypeStruct((N, HID), jnp.float32),
            jax.ShapeDtypeStruct((N, TS), jnp.float32),
            jax.ShapeDtypeStruct((N, 16), jnp.float32),
        ),
    )(x, W_res, b_res.reshape(1, -1), W1, As, Ad)


def _edge_body(tsrc_hbm, tdst_hbm, src_hbm, dst_hbm, zeros_hbm, out_hbm,
               sidx, didx, rsrc, rdst, obuf, acc, sem1, sem2):
    c = lax.axis_index("c")
    s = lax.axis_index("s")
    wid = s * _NC + c
    pltpu.sync_copy(zeros_hbm, acc.at[pl.ds(s * _RPT, _RPT)])

    @pl.when(s == _NS - 1)
    def _zero_tail():
        pltpu.sync_copy(zeros_hbm.at[pl.ds(0, _TAIL)],
                        acc.at[pl.ds(_NS * _RPT, _TAIL)])

    plsc.subcore_barrier()
    base = wid * _EW

    def chunk(i, carry):
        off = base + i * _CH
        pltpu.sync_copy(src_hbm.at[pl.ds(off, _CH)], sidx)
        pltpu.sync_copy(dst_hbm.at[pl.ds(off, _CH)], didx)
        d1 = pltpu.async_copy(tsrc_hbm.at[sidx], rsrc, sem1)
        d2 = pltpu.async_copy(tdst_hbm.at[didx], rdst, sem2)
        d1.wait()
        d2.wait()

        def edge(e, carry2):
            va = rsrc[e, pl.ds(HID, 16)]
            vb = rdst[e, pl.ds(0, 16)]
            sv = va + vb
            ex = jnp.exp(jnp.where(sv < 0, sv * 0.2, sv))
            obuf[e, pl.ds(HID, 16)] = ex
            for h in range(HEADS):
                obuf[e, pl.ds(16 * h, 16)] = rsrc[e, pl.ds(16 * h, 16)] * ex[h]
            return carry2

        lax.fori_loop(0, _CH, edge, 0)
        pltpu.sync_copy(obuf, acc.at[didx], add=True)
        return carry

    lax.fori_loop(0, _NCHUNK, chunk, 0)
    plsc.subcore_barrier()
    pltpu.sync_copy(acc.at[pl.ds(s * _RPT, _RPT)],
                    out_hbm.at[c, pl.ds(s * _RPT, _RPT)])

    @pl.when(s == _NS - 1)
    def _dump_tail():
        pltpu.sync_copy(acc.at[pl.ds(_NS * _RPT, _TAIL)],
                        out_hbm.at[c, pl.ds(_NS * _RPT, _TAIL)])


_edge_pass = pl.kernel(
    _edge_body,
    out_type=jax.ShapeDtypeStruct((_NC, N, TS), jnp.float32),
    mesh=plsc.VectorSubcoreMesh(core_axis_name="c", subcore_axis_name="s",
                                num_cores=_NC, num_subcores=_NS),
    scratch_types=[
        pltpu.VMEM((_CH,), jnp.int32),
        pltpu.VMEM((_CH,), jnp.int32),
        pltpu.VMEM((_CH, TS), jnp.float32),
        pltpu.VMEM((_CH, 16), jnp.float32),
        pltpu.VMEM((_CH, TS), jnp.float32),
        pltpu.VMEM_SHARED((N, TS), jnp.float32),
        pltpu.SemaphoreType.DMA,
        pltpu.SemaphoreType.DMA,
    ],
    compiler_params=pltpu.CompilerParams(use_tc_tiling_on_sc=False),
)


def _gat_finalize(acc0, acc1, t_src, t_dst, b16, bconv, g, be):
    acc = acc0 + acc1
    h = t_src[:, :HID]
    es = t_src[:, HID:HID + 8] + t_dst[:, :8]
    exs = jnp.exp(jnp.where(es < 0, es * 0.2, es))
    exs_b = jnp.dot(exs, b16, preferred_element_type=jnp.float32)
    num = acc[:, :HID] + exs_b * h
    den = acc[:, HID:HID + 8] + exs
    den_b = jnp.dot(den, b16, preferred_element_type=jnp.float32)
    gat = num / (den_b + 1e-16) + bconv
    m = jnp.mean(gat, axis=0, keepdims=True)
    v = jnp.mean((gat - m) ** 2, axis=0, keepdims=True)
    gn = (gat - m) / jnp.sqrt(v + 1e-5) * g + be
    return jnp.where(gn > 0, gn, jnp.exp(gn) - 1.0)


def _mid_body(acc_ref, tsrc1_ref, tdst1_ref, b16_ref, bconv1_ref, g1_ref,
              be1_ref, w2_ref, asm2_ref, adm2_ref, tsrc2_ref, tdst2_ref):
    f32 = jnp.float32
    h1 = _gat_finalize(acc_ref[0], acc_ref[1], tsrc1_ref[...], tdst1_ref[...],
                       b16_ref[...], bconv1_ref[...], g1_ref[...], be1_ref[...])
    h2 = jnp.dot(h1, w2_ref[...].T, preferred_element_type=f32)
    asrc = jnp.dot(h2, asm2_ref[...], preferred_element_type=f32)
    adst = jnp.dot(h2, adm2_ref[...], preferred_element_type=f32)
    z8 = jnp.zeros((N, 8), f32)
    tsrc2_ref[...] = jnp.concatenate([h2, asrc, z8], axis=1)
    tdst2_ref[...] = jnp.concatenate([adst, z8], axis=1)


def _mid(acc, tsrc1, tdst1, B16, bconv1, g1, be1, W2, As2, Ad2):
    return pl.pallas_call(
        _mid_body,
        out_shape=(
            jax.ShapeDtypeStruct((N, TS), jnp.float32),
            jax.ShapeDtypeStruct((N, 16), jnp.float32),
        ),
    )(acc, tsrc1, tdst1, B16, bconv1.reshape(1, -1), g1.reshape(1, -1),
      be1.reshape(1, -1), W2, As2, Ad2)


def _final_body(acc_ref, tsrc2_ref, tdst2_ref, xp_ref, b16_ref, bconv2_ref,
                g2_ref, be2_ref, wc1_ref, bc1_ref, gcn_ref, bcn_ref, wc2_ref,
                bc2_ref, out_ref):
    f32 = jnp.float32
    h2 = _gat_finalize(acc_ref[0], acc_ref[1], tsrc2_ref[...], tdst2_ref[...],
                       b16_ref[...], bconv2_ref[...], g2_ref[...], be2_ref[...])
    h2 = h2 + xp_ref[...]
    c = jnp.dot(h2, wc1_ref[...].T, preferred_element_type=f32) + bc1_ref[...]
    m = jnp.mean(c, axis=0, keepdims=True)
    v = jnp.mean((c - m) ** 2, axis=0, keepdims=True)
    c = (c - m) / jnp.sqrt(v + 1e-5) * gcn_ref[...] + bcn_ref[...]
    c = jnp.maximum(c, 0.0)
    logits = jnp.dot(c, wc2_ref[...].T, preferred_element_type=f32)
    logits = logits + bc2_ref[...]
    mx = jnp.max(logits, axis=1, keepdims=True)
    sh = logits - mx
    lse = jnp.log(jnp.sum(jnp.exp(sh), axis=1, keepdims=True))
    out_ref[...] = sh - lse


def _final(acc, tsrc2, tdst2, x_p, B16, bconv2, g2, be2, Wc1, bc1, gcn, bcn,
           Wc2, bc2):
    return pl.pallas_call(
        _final_body,
        out_shape=jax.ShapeDtypeStruct((N, OUT), jnp.float32),
    )(acc, tsrc2, tdst2, x_p, B16, bconv2.reshape(1, -1), g2.reshape(1, -1),
      be2.reshape(1, -1), Wc1, bc1.reshape(1, -1), gcn.reshape(1, -1),
      bcn.reshape(1, -1), Wc2, bc2.reshape(1, -1))


def _att_mat(a):
    return (a[:, :, None] * jnp.eye(HEADS, dtype=a.dtype)[:, None, :]
            ).reshape(HID, HEADS)


def kernel(x, edge_index, W_res, b_res, W1, as1, ad1, bconv1, g1, be1,
           W2, as2, ad2, bconv2, g2, be2, Wc1, bc1, gcn, bcn, Wc2, bc2):
    src = edge_index[0].astype(jnp.int32)
    dst = edge_index[1].astype(jnp.int32)
    B16 = jnp.kron(jnp.eye(HEADS, dtype=jnp.float32),
                   jnp.ones((1, HD), dtype=jnp.float32))
    zeros_tile = jnp.zeros((_RPT, TS), jnp.float32)

    x_p, tsrc1, tdst1 = _prep(x, W_res, b_res, W1, _att_mat(as1), _att_mat(ad1))
    acc1 = _edge_pass(tsrc1, tdst1, src, dst, zeros_tile)
    tsrc2, tdst2 = _mid(acc1, tsrc1, tdst1, B16, bconv1, g1, be1,
                        W2, _att_mat(as2), _att_mat(ad2))
    acc2 = _edge_pass(tsrc2, tdst2, src, dst, zeros_tile)
    return _final(acc2, tsrc2, tdst2, x_p, B16, bconv2, g2, be2,
                  Wc1, bc1, gcn, bcn, Wc2, bc2)


# speedup vs baseline: 115.6082x; 1.6219x over previous
import functools

import jax
import jax.numpy as jnp
from jax import lax
from jax.experimental import pallas as pl
from jax.experimental.pallas import tpu as pltpu
from jax.experimental.pallas import tpu_sc as plsc

N = 10000
E = 320000
HEADS = 8
HD = 16
HID = HEADS * HD
OUT = 64
TS = 144

_NC = 2
_NS = 16
_NW = _NC * _NS
_EW = E // _NW
_CH = 40
_NCHUNK = _EW // _CH
_HB = _NCHUNK // 2
_RPT = 624
_TAIL = N - _NS * _RPT


def _prep_body(x_ref, wres_ref, bres_ref, w1_ref, asm_ref, adm_ref,
               xp_ref, tsrc_ref, tdst_ref):
    f32 = jnp.float32
    xp = jnp.dot(x_ref[...], wres_ref[...].T, preferred_element_type=f32)
    xp = xp + bres_ref[...]
    xp_ref[...] = xp
    h = jnp.dot(xp, w1_ref[...].T, preferred_element_type=f32)
    asrc = jnp.dot(h, asm_ref[...], preferred_element_type=f32)
    adst = jnp.dot(h, adm_ref[...], preferred_element_type=f32)
    z8 = jnp.zeros((N, 8), f32)
    tsrc_ref[...] = jnp.concatenate([h, asrc, z8], axis=1)
    tdst_ref[...] = jnp.concatenate([adst, z8], axis=1)


def _prep(x, W_res, b_res, W1, As, Ad):
    return pl.pallas_call(
        _prep_body,
        out_shape=(
            jax.ShapeDtypeStruct((N, HID), jnp.float32),
            jax.ShapeDtypeStruct((N, TS), jnp.float32),
            jax.ShapeDtypeStruct((N, 16), jnp.float32),
        ),
    )(x, W_res, b_res.reshape(1, -1), W1, As, Ad)


def _edge_body(tsrc_hbm, tdst_hbm, src_hbm, dst_hbm, zeros_hbm, out_hbm,
               sidx, didx, rsrc0, rdst0, obuf0, rsrc1, rdst1, obuf1,
               acc, sg0, sg1):
    c = lax.axis_index("c")
    s = lax.axis_index("s")
    wid = s * _NC + c
    pltpu.sync_copy(zeros_hbm, acc.at[pl.ds(s * _RPT, _RPT)])

    @pl.when(s == _NS - 1)
    def _zero_tail():
        pltpu.sync_copy(zeros_hbm.at[pl.ds(0, _TAIL)],
                        acc.at[pl.ds(_NS * _RPT, _TAIL)])

    plsc.subcore_barrier()

    def issue(i, rsrc, rdst, sem):
        d1 = pltpu.async_copy(tsrc_hbm.at[sidx.at[i]], rsrc, sem)
        d2 = pltpu.async_copy(tdst_hbm.at[didx.at[i]], rdst, sem)
        return d1, d2

    def consume(i, rsrc, rdst, obuf, descs):
        descs[0].wait()
        descs[1].wait()

        def edge(e, carry2):
            va = rsrc[e, pl.ds(HID, 16)]
            vb = rdst[e, pl.ds(0, 16)]
            sv = va + vb
            ex = jnp.exp(jnp.where(sv < 0, sv * 0.2, sv))
            obuf[e, pl.ds(HID, 16)] = ex
            for h in range(HEADS):
                obuf[e, pl.ds(16 * h, 16)] = rsrc[e, pl.ds(16 * h, 16)] * ex[h]
            return carry2

        lax.fori_loop(0, _CH, edge, 0)
        pltpu.sync_copy(obuf, acc.at[didx.at[i]], add=True)

    for half in range(2):
        hb = wid * _NCHUNK + half * _HB
        pltpu.sync_copy(src_hbm.at[pl.ds(hb, _HB)], sidx)
        pltpu.sync_copy(dst_hbm.at[pl.ds(hb, _HB)], didx)

        d0 = issue(0, rsrc0, rdst0, sg0)

        def stage(k, carry):
            g0 = 2 * k
            d1 = issue(g0 + 1, rsrc1, rdst1, sg1)
            consume(g0, rsrc0, rdst0, obuf0, d0)
            issue(g0 + 2, rsrc0, rdst0, sg0)
            consume(g0 + 1, rsrc1, rdst1, obuf1, d1)
            return carry

        lax.fori_loop(0, (_HB - 1) // 2, stage, 0)
        consume(_HB - 1, rsrc0, rdst0, obuf0, d0)

    plsc.subcore_barrier()
    pltpu.sync_copy(acc.at[pl.ds(s * _RPT, _RPT)],
                    out_hbm.at[c, pl.ds(s * _RPT, _RPT)])

    @pl.when(s == _NS - 1)
    def _dump_tail():
        pltpu.sync_copy(acc.at[pl.ds(_NS * _RPT, _TAIL)],
                        out_hbm.at[c, pl.ds(_NS * _RPT, _TAIL)])


_edge_pass = pl.kernel(
    _edge_body,
    out_type=jax.ShapeDtypeStruct((_NC, N, TS), jnp.float32),
    mesh=plsc.VectorSubcoreMesh(core_axis_name="c", subcore_axis_name="s",
                                num_cores=_NC, num_subcores=_NS),
    scratch_types=[
        pltpu.VMEM((_HB, _CH), jnp.int32),
        pltpu.VMEM((_HB, _CH), jnp.int32),
        pltpu.VMEM((_CH, TS), jnp.float32),
        pltpu.VMEM((_CH, 16), jnp.float32),
        pltpu.VMEM((_CH, TS), jnp.float32),
        pltpu.VMEM((_CH, TS), jnp.float32),
        pltpu.VMEM((_CH, 16), jnp.float32),
        pltpu.VMEM((_CH, TS), jnp.float32),
        pltpu.VMEM_SHARED((N, TS), jnp.float32),
        pltpu.SemaphoreType.DMA,
        pltpu.SemaphoreType.DMA,
    ],
    compiler_params=pltpu.CompilerParams(use_tc_tiling_on_sc=False),
)


def _gat_finalize(acc0, acc1, t_src, t_dst, b16, bconv, g, be):
    acc = acc0 + acc1
    h = t_src[:, :HID]
    es = t_src[:, HID:HID + 8] + t_dst[:, :8]
    exs = jnp.exp(jnp.where(es < 0, es * 0.2, es))
    exs_b = jnp.dot(exs, b16, preferred_element_type=jnp.float32)
    num = acc[:, :HID] + exs_b * h
    den = acc[:, HID:HID + 8] + exs
    den_b = jnp.dot(den, b16, preferred_element_type=jnp.float32)
    gat = num / (den_b + 1e-16) + bconv
    m = jnp.mean(gat, axis=0, keepdims=True)
    v = jnp.mean((gat - m) ** 2, axis=0, keepdims=True)
    gn = (gat - m) / jnp.sqrt(v + 1e-5) * g + be
    return jnp.where(gn > 0, gn, jnp.exp(gn) - 1.0)


def _mid_body(acc_ref, tsrc1_ref, tdst1_ref, b16_ref, bconv1_ref, g1_ref,
              be1_ref, w2_ref, asm2_ref, adm2_ref, tsrc2_ref, tdst2_ref):
    f32 = jnp.float32
    h1 = _gat_finalize(acc_ref[0], acc_ref[1], tsrc1_ref[...], tdst1_ref[...],
                       b16_ref[...], bconv1_ref[...], g1_ref[...], be1_ref[...])
    h2 = jnp.dot(h1, w2_ref[...].T, preferred_element_type=f32)
    asrc = jnp.dot(h2, asm2_ref[...], preferred_element_type=f32)
    adst = jnp.dot(h2, adm2_ref[...], preferred_element_type=f32)
    z8 = jnp.zeros((N, 8), f32)
    tsrc2_ref[...] = jnp.concatenate([h2, asrc, z8], axis=1)
    tdst2_ref[...] = jnp.concatenate([adst, z8], axis=1)


def _mid(acc, tsrc1, tdst1, B16, bconv1, g1, be1, W2, As2, Ad2):
    return pl.pallas_call(
        _mid_body,
        out_shape=(
            jax.ShapeDtypeStruct((N, TS), jnp.float32),
            jax.ShapeDtypeStruct((N, 16), jnp.float32),
        ),
    )(acc, tsrc1, tdst1, B16, bconv1.reshape(1, -1), g1.reshape(1, -1),
      be1.reshape(1, -1), W2, As2, Ad2)


def _final_body(acc_ref, tsrc2_ref, tdst2_ref, xp_ref, b16_ref, bconv2_ref,
                g2_ref, be2_ref, wc1_ref, bc1_ref, gcn_ref, bcn_ref, wc2_ref,
                bc2_ref, out_ref):
    f32 = jnp.float32
    h2 = _gat_finalize(acc_ref[0], acc_ref[1], tsrc2_ref[...], tdst2_ref[...],
                       b16_ref[...], bconv2_ref[...], g2_ref[...], be2_ref[...])
    h2 = h2 + xp_ref[...]
    c = jnp.dot(h2, wc1_ref[...].T, preferred_element_type=f32) + bc1_ref[...]
    m = jnp.mean(c, axis=0, keepdims=True)
    v = jnp.mean((c - m) ** 2, axis=0, keepdims=True)
    c = (c - m) / jnp.sqrt(v + 1e-5) * gcn_ref[...] + bcn_ref[...]
    c = jnp.maximum(c, 0.0)
    logits = jnp.dot(c, wc2_ref[...].T, preferred_element_type=f32)
    logits = logits + bc2_ref[...]
    mx = jnp.max(logits, axis=1, keepdims=True)
    sh = logits - mx
    lse = jnp.log(jnp.sum(jnp.exp(sh), axis=1, keepdims=True))
    out_ref[...] = sh - lse


def _final(acc, tsrc2, tdst2, x_p, B16, bconv2, g2, be2, Wc1, bc1, gcn, bcn,
           Wc2, bc2):
    return pl.pallas_call(
        _final_body,
        out_shape=jax.ShapeDtypeStruct((N, OUT), jnp.float32),
    )(acc, tsrc2, tdst2, x_p, B16, bconv2.reshape(1, -1), g2.reshape(1, -1),
      be2.reshape(1, -1), Wc1, bc1.reshape(1, -1), gcn.reshape(1, -1),
      bcn.reshape(1, -1), Wc2, bc2.reshape(1, -1))


def _att_mat(a):
    return (a[:, :, None] * jnp.eye(HEADS, dtype=a.dtype)[:, None, :]
            ).reshape(HID, HEADS)


def kernel(x, edge_index, W_res, b_res, W1, as1, ad1, bconv1, g1, be1,
           W2, as2, ad2, bconv2, g2, be2, Wc1, bc1, gcn, bcn, Wc2, bc2):
    src = edge_index[0].astype(jnp.int32).reshape(E // _CH, _CH)
    dst = edge_index[1].astype(jnp.int32).reshape(E // _CH, _CH)
    B16 = jnp.kron(jnp.eye(HEADS, dtype=jnp.float32),
                   jnp.ones((1, HD), dtype=jnp.float32))
    zeros_tile = jnp.zeros((_RPT, TS), jnp.float32)

    x_p, tsrc1, tdst1 = _prep(x, W_res, b_res, W1, _att_mat(as1), _att_mat(ad1))
    acc1 = _edge_pass(tsrc1, tdst1, src, dst, zeros_tile)
    tsrc2, tdst2 = _mid(acc1, tsrc1, tdst1, B16, bconv1, g1, be1,
                        W2, _att_mat(as2), _att_mat(ad2))
    acc2 = _edge_pass(tsrc2, tdst2, src, dst, zeros_tile)
    return _final(acc2, tsrc2, tdst2, x_p, B16, bconv2, g2, be2,
                  Wc1, bc1, gcn, bcn, Wc2, bc2)


# speedup vs baseline: 129.5220x; 1.1204x over previous
import functools

import jax
import jax.numpy as jnp
from jax import lax
from jax.experimental import pallas as pl
from jax.experimental.pallas import tpu as pltpu
from jax.experimental.pallas import tpu_sc as plsc

N = 10000
E = 320000
HEADS = 8
HD = 16
HID = HEADS * HD
OUT = 64
TS = 144

_NC = 2
_NS = 16
_NW = _NC * _NS
_EW = E // _NW
_CH = 40
_NCHUNK = _EW // _CH
_HB = _NCHUNK // 2
_RPT = 624
_TAIL = N - _NS * _RPT


def _prep_body(x_ref, wres_ref, bres_ref, w1_ref, asm_ref, adm_ref,
               xp_ref, tsrc_ref, tdst_ref):
    f32 = jnp.float32
    xp = jnp.dot(x_ref[...], wres_ref[...].T, preferred_element_type=f32)
    xp = xp + bres_ref[...]
    xp_ref[...] = xp
    h = jnp.dot(xp, w1_ref[...].T, preferred_element_type=f32)
    asrc = jnp.dot(h, asm_ref[...], preferred_element_type=f32)
    adst = jnp.dot(h, adm_ref[...], preferred_element_type=f32)
    z8 = jnp.zeros((N, 8), f32)
    tsrc_ref[...] = jnp.concatenate([h, asrc, z8], axis=1)
    tdst_ref[...] = jnp.concatenate([adst, z8], axis=1)


def _prep(x, W_res, b_res, W1, As, Ad):
    return pl.pallas_call(
        _prep_body,
        out_shape=(
            jax.ShapeDtypeStruct((N, HID), jnp.float32),
            jax.ShapeDtypeStruct((N, TS), jnp.float32),
            jax.ShapeDtypeStruct((N, 16), jnp.float32),
        ),
    )(x, W_res, b_res.reshape(1, -1), W1, As, Ad)


def _edge_body(tsrc_hbm, tdst_hbm, src_hbm, dst_hbm, zeros_hbm, out_hbm,
               sidx, didx, rsrc0, rdst0, obuf0, rsrc1, rdst1, obuf1,
               acc, sg0, sg1, ss0, ss1):
    c = lax.axis_index("c")
    s = lax.axis_index("s")
    wid = s * _NC + c
    pltpu.sync_copy(zeros_hbm, acc.at[pl.ds(s * _RPT, _RPT)])

    @pl.when(s == _NS - 1)
    def _zero_tail():
        pltpu.sync_copy(zeros_hbm.at[pl.ds(0, _TAIL)],
                        acc.at[pl.ds(_NS * _RPT, _TAIL)])

    plsc.subcore_barrier()

    def issue(i, rsrc, rdst, sem):
        d1 = pltpu.async_copy(tsrc_hbm.at[sidx.at[i]], rsrc, sem)
        d2 = pltpu.async_copy(tdst_hbm.at[didx.at[i]], rdst, sem)
        return d1, d2

    def consume(i, rsrc, rdst, obuf, descs, ssem):
        descs[0].wait()
        descs[1].wait()
        pltpu.make_async_copy(obuf, acc.at[didx.at[i]], ssem).wait()

        def edge(e, carry2):
            va = rsrc[e, pl.ds(HID, 16)]
            vb = rdst[e, pl.ds(0, 16)]
            sv = va + vb
            ex = jnp.exp(jnp.where(sv < 0, sv * 0.2, sv))
            obuf[e, pl.ds(HID, 16)] = ex
            for h in range(HEADS):
                obuf[e, pl.ds(16 * h, 16)] = rsrc[e, pl.ds(16 * h, 16)] * ex[h]
            return carry2

        lax.fori_loop(0, _CH, edge, 0)
        pltpu.async_copy(obuf, acc.at[didx.at[i]], ssem, add=True)

    for half in range(2):
        hb = wid * _NCHUNK + half * _HB
        pltpu.sync_copy(src_hbm.at[pl.ds(hb, _HB)], sidx)
        pltpu.sync_copy(dst_hbm.at[pl.ds(hb, _HB)], didx)
        pltpu.sync_copy(zeros_hbm.at[pl.ds(0, _CH)], obuf0)
        pltpu.sync_copy(zeros_hbm.at[pl.ds(0, _CH)], obuf1)
        pltpu.async_copy(obuf0, acc.at[didx.at[0]], ss0, add=True)
        pltpu.async_copy(obuf1, acc.at[didx.at[0]], ss1, add=True)

        d0 = issue(0, rsrc0, rdst0, sg0)

        def stage(k, carry):
            g0 = 2 * k
            d1 = issue(g0 + 1, rsrc1, rdst1, sg1)
            consume(g0, rsrc0, rdst0, obuf0, d0, ss0)
            issue(g0 + 2, rsrc0, rdst0, sg0)
            consume(g0 + 1, rsrc1, rdst1, obuf1, d1, ss1)
            return carry

        lax.fori_loop(0, (_HB - 1) // 2, stage, 0)
        consume(_HB - 1, rsrc0, rdst0, obuf0, d0, ss0)
        pltpu.make_async_copy(obuf0, acc.at[didx.at[0]], ss0).wait()
        pltpu.make_async_copy(obuf1, acc.at[didx.at[0]], ss1).wait()

    plsc.subcore_barrier()
    pltpu.sync_copy(acc.at[pl.ds(s * _RPT, _RPT)],
                    out_hbm.at[c, pl.ds(s * _RPT, _RPT)])

    @pl.when(s == _NS - 1)
    def _dump_tail():
        pltpu.sync_copy(acc.at[pl.ds(_NS * _RPT, _TAIL)],
                        out_hbm.at[c, pl.ds(_NS * _RPT, _TAIL)])


_edge_pass = pl.kernel(
    _edge_body,
    out_type=jax.ShapeDtypeStruct((_NC, N, TS), jnp.float32),
    mesh=plsc.VectorSubcoreMesh(core_axis_name="c", subcore_axis_name="s",
                                num_cores=_NC, num_subcores=_NS),
    scratch_types=[
        pltpu.VMEM((_HB, _CH), jnp.int32),
        pltpu.VMEM((_HB, _CH), jnp.int32),
        pltpu.VMEM((_CH, TS), jnp.float32),
        pltpu.VMEM((_CH, 16), jnp.float32),
        pltpu.VMEM((_CH, TS), jnp.float32),
        pltpu.VMEM((_CH, TS), jnp.float32),
        pltpu.VMEM((_CH, 16), jnp.float32),
        pltpu.VMEM((_CH, TS), jnp.float32),
        pltpu.VMEM_SHARED((N, TS), jnp.float32),
        pltpu.SemaphoreType.DMA,
        pltpu.SemaphoreType.DMA,
        pltpu.SemaphoreType.DMA,
        pltpu.SemaphoreType.DMA,
    ],
    compiler_params=pltpu.CompilerParams(use_tc_tiling_on_sc=False),
)


def _gat_finalize(acc0, acc1, t_src, t_dst, b16, bconv, g, be):
    acc = acc0 + acc1
    h = t_src[:, :HID]
    es = t_src[:, HID:HID + 8] + t_dst[:, :8]
    exs = jnp.exp(jnp.where(es < 0, es * 0.2, es))
    exs_b = jnp.dot(exs, b16, preferred_element_type=jnp.float32)
    num = acc[:, :HID] + exs_b * h
    den = acc[:, HID:HID + 8] + exs
    den_b = jnp.dot(den, b16, preferred_element_type=jnp.float32)
    gat = num / (den_b + 1e-16) + bconv
    m = jnp.mean(gat, axis=0, keepdims=True)
    v = jnp.mean((gat - m) ** 2, axis=0, keepdims=True)
    gn = (gat - m) / jnp.sqrt(v + 1e-5) * g + be
    return jnp.where(gn > 0, gn, jnp.exp(gn) - 1.0)


def _mid_body(acc_ref, tsrc1_ref, tdst1_ref, b16_ref, bconv1_ref, g1_ref,
              be1_ref, w2_ref, asm2_ref, adm2_ref, tsrc2_ref, tdst2_ref):
    f32 = jnp.float32
    h1 = _gat_finalize(acc_ref[0], acc_ref[1], tsrc1_ref[...], tdst1_ref[...],
                       b16_ref[...], bconv1_ref[...], g1_ref[...], be1_ref[...])
    h2 = jnp.dot(h1, w2_ref[...].T, preferred_element_type=f32)
    asrc = jnp.dot(h2, asm2_ref[...], preferred_element_type=f32)
    adst = jnp.dot(h2, adm2_ref[...], preferred_element_type=f32)
    z8 = jnp.zeros((N, 8), f32)
    tsrc2_ref[...] = jnp.concatenate([h2, asrc, z8], axis=1)
    tdst2_ref[...] = jnp.concatenate([adst, z8], axis=1)


def _mid(acc, tsrc1, tdst1, B16, bconv1, g1, be1, W2, As2, Ad2):
    return pl.pallas_call(
        _mid_body,
        out_shape=(
            jax.ShapeDtypeStruct((N, TS), jnp.float32),
            jax.ShapeDtypeStruct((N, 16), jnp.float32),
        ),
    )(acc, tsrc1, tdst1, B16, bconv1.reshape(1, -1), g1.reshape(1, -1),
      be1.reshape(1, -1), W2, As2, Ad2)


def _final_body(acc_ref, tsrc2_ref, tdst2_ref, xp_ref, b16_ref, bconv2_ref,
                g2_ref, be2_ref, wc1_ref, bc1_ref, gcn_ref, bcn_ref, wc2_ref,
                bc2_ref, out_ref):
    f32 = jnp.float32
    h2 = _gat_finalize(acc_ref[0], acc_ref[1], tsrc2_ref[...], tdst2_ref[...],
                       b16_ref[...], bconv2_ref[...], g2_ref[...], be2_ref[...])
    h2 = h2 + xp_ref[...]
    c = jnp.dot(h2, wc1_ref[...].T, preferred_element_type=f32) + bc1_ref[...]
    m = jnp.mean(c, axis=0, keepdims=True)
    v = jnp.mean((c - m) ** 2, axis=0, keepdims=True)
    c = (c - m) / jnp.sqrt(v + 1e-5) * gcn_ref[...] + bcn_ref[...]
    c = jnp.maximum(c, 0.0)
    logits = jnp.dot(c, wc2_ref[...].T, preferred_element_type=f32)
    logits = logits + bc2_ref[...]
    mx = jnp.max(logits, axis=1, keepdims=True)
    sh = logits - mx
    lse = jnp.log(jnp.sum(jnp.exp(sh), axis=1, keepdims=True))
    out_ref[...] = sh - lse


def _final(acc, tsrc2, tdst2, x_p, B16, bconv2, g2, be2, Wc1, bc1, gcn, bcn,
           Wc2, bc2):
    return pl.pallas_call(
        _final_body,
        out_shape=jax.ShapeDtypeStruct((N, OUT), jnp.float32),
    )(acc, tsrc2, tdst2, x_p, B16, bconv2.reshape(1, -1), g2.reshape(1, -1),
      be2.reshape(1, -1), Wc1, bc1.reshape(1, -1), gcn.reshape(1, -1),
      bcn.reshape(1, -1), Wc2, bc2.reshape(1, -1))


def _att_mat(a):
    return (a[:, :, None] * jnp.eye(HEADS, dtype=a.dtype)[:, None, :]
            ).reshape(HID, HEADS)


def kernel(x, edge_index, W_res, b_res, W1, as1, ad1, bconv1, g1, be1,
           W2, as2, ad2, bconv2, g2, be2, Wc1, bc1, gcn, bcn, Wc2, bc2):
    src = edge_index[0].astype(jnp.int32).reshape(E // _CH, _CH)
    dst = edge_index[1].astype(jnp.int32).reshape(E // _CH, _CH)
    B16 = jnp.kron(jnp.eye(HEADS, dtype=jnp.float32),
                   jnp.ones((1, HD), dtype=jnp.float32))
    zeros_tile = jnp.zeros((_RPT, TS), jnp.float32)

    x_p, tsrc1, tdst1 = _prep(x, W_res, b_res, W1, _att_mat(as1), _att_mat(ad1))
    acc1 = _edge_pass(tsrc1, tdst1, src, dst, zeros_tile)
    tsrc2, tdst2 = _mid(acc1, tsrc1, tdst1, B16, bconv1, g1, be1,
                        W2, _att_mat(as2), _att_mat(ad2))
    acc2 = _edge_pass(tsrc2, tdst2, src, dst, zeros_tile)
    return _final(acc2, tsrc2, tdst2, x_p, B16, bconv2, g2, be2,
                  Wc1, bc1, gcn, bcn, Wc2, bc2)
